# baseline (speedup 1.0000x reference)
import functools

import jax
import jax.numpy as jnp
from jax import lax
from jax.experimental import pallas as pl
from jax.experimental.pallas import tpu as pltpu
from jax.experimental.pallas import tpu_sc as plsc

D = 256
SEG = 16
N = 32768
NC = 2
NS = 16
NW = NC * NS
RPW = N // NW
CHUNK = 128
NCHUNK = RPW // CHUNK
L = 16
CPR = D // L
NEG = -1e30

_mesh = plsc.VectorSubcoreMesh(
    core_axis_name="c", subcore_axis_name="s", num_cores=NC, num_subcores=NS)

_f32 = jnp.float32
_i32 = jnp.int32

_sc_params = pltpu.CompilerParams(needs_layout_passes=False)


def _lane_iota():
  return lax.iota(_i32, L)


_GATHER_DNUMS = lax.GatherDimensionNumbers(
    offset_dims=(), collapsed_slice_dims=(0,), start_index_map=(0,))


def _shuffle(vec, idx):
  return lax.gather(vec, idx[:, None], _GATHER_DNUMS, (1,),
                    mode=lax.GatherScatterMode.PROMISE_IN_BOUNDS)


def _allsum(vec):
  iota = _lane_iota()
  for k in (1, 2, 4, 8):
    vec = vec + _shuffle(vec, jnp.bitwise_xor(iota, k))
  return vec


def _stage_offsets(off_hbm, offs_v, offs_s):
  pltpu.sync_copy(off_hbm, offs_v)
  v0 = offs_v[pl.ds(0, L)]
  v1 = offs_v[pl.ds(L, L)]
  for s in range(L):
    offs_s[s] = v0[s]
  offs_s[L] = v1[0]


@functools.partial(
    pl.kernel,
    out_type=[jax.ShapeDtypeStruct((NW, SEG, D), _f32),
              jax.ShapeDtypeStruct((NW, SEG, D), _f32)],
    mesh=_mesh,
    scratch_types=[
        pltpu.VMEM((2, CHUNK, D), _f32),
        pltpu.VMEM((32,), _i32),
        pltpu.VMEM((SEG, D), _f32),
        pltpu.SMEM((SEG + 1,), _i32),
        pltpu.SemaphoreType.DMA,
    ],
    compiler_params=_sc_params,
)
def _sc_pass1(hA, offA, hB, offB, outA, outB, buf, offs_v, acc, offs_s, sem):
  wid = lax.axis_index("s") * NC + lax.axis_index("c")

  for h, off_hbm, out in ((hA, offA, outA), (hB, offB, outB)):
    _stage_offsets(off_hbm, offs_v, offs_s)
    z = jnp.zeros((L,), _f32)

    def zbody(s, _):
      for c in range(CPR):
        acc[s, pl.ds(c * L, L)] = z
      return 0

    lax.fori_loop(0, SEG, zbody, 0)
    base0 = wid * RPW
    pltpu.make_async_copy(h.at[pl.ds(base0, CHUNK)], buf.at[0], sem).start()

    def chunk_body(k, _):
      par = lax.bitwise_and(k, 1)
      base = base0 + k * CHUNK
      pltpu.make_async_copy(h.at[pl.ds(0, CHUNK)], buf.at[0], sem).wait()

      @pl.when(k + 1 < NCHUNK)
      def _():
        pltpu.make_async_copy(
            h.at[pl.ds(base + CHUNK, CHUNK)],
            buf.at[lax.bitwise_and(k + 1, 1)], sem).start()

      def seg_body(s, _):
        lo = jnp.clip(offs_s[s] - base, 0, CHUNK)
        hi = jnp.clip(offs_s[s + 1] - base, 0, CHUNK)

        @pl.when(hi > lo)
        def _():
          def row_body(r, carry):
            return tuple(
                carry[c] + buf[par, r, pl.ds(c * L, L)] for c in range(CPR))

          def quad_body(j, carry):
            r0 = lo + j * 4
            for u in range(4):
              carry = row_body(r0 + u, carry)
            return carry

          n4 = lax.bitwise_and(hi - lo, -4)
          zeros = tuple(jnp.zeros((L,), _f32) for _ in range(CPR))
          tot = lax.fori_loop(0, lax.shift_right_logical(n4, 2),
                              quad_body, zeros)
          tot = lax.fori_loop(lo + n4, hi, row_body, tot)
          for c in range(CPR):
            acc[s, pl.ds(c * L, L)] += tot[c]

        return 0

      lax.fori_loop(0, SEG, seg_body, 0)
      return 0

    lax.fori_loop(0, NCHUNK, chunk_body, 0)
    pltpu.sync_copy(acc, out.at[wid])


def _tc_proj_body(psA, psB, cA, cB, w, projA, projB):
  wmat = w[...]
  for ps, cnt, out in ((psA, cA, projA), (psB, cB, projB)):
    sums = jnp.sum(ps[...], axis=0)
    mean = sums / jnp.maximum(cnt[...], 1.0)
    out[...] = jnp.dot(mean, wmat, preferred_element_type=_f32)


def _tc_proj(psA, psB, cA, cB, w):
  return pl.pallas_call(
      _tc_proj_body,
      out_shape=[jax.ShapeDtypeStruct((SEG, D), _f32),
                 jax.ShapeDtypeStruct((SEG, D), _f32)],
  )(psA, psB, cA, cB, w)


@functools.partial(
    pl.kernel,
    out_type=[jax.ShapeDtypeStruct((NW, SEG), _f32),
              jax.ShapeDtypeStruct((NW, SEG), _f32),
              jax.ShapeDtypeStruct((NW, SEG, D), _f32),
              jax.ShapeDtypeStruct((NW, SEG), _f32),
              jax.ShapeDtypeStruct((NW, SEG), _f32),
              jax.ShapeDtypeStruct((NW, SEG, D), _f32)],
    mesh=_mesh,
    scratch_types=[
        pltpu.VMEM((2, CHUNK, D), _f32),
        pltpu.VMEM((32,), _i32),
        pltpu.VMEM((SEG, D), _f32),
        pltpu.VMEM((SEG, D), _f32),
        pltpu.VMEM((SEG * L,), _f32),
        pltpu.VMEM((SEG * L,), _f32),
        pltpu.VMEM(((CHUNK + L) * L,), _f32),
        pltpu.VMEM((L,), _f32),
        pltpu.VMEM((L,), _f32),
        pltpu.SMEM((SEG + 1,), _i32),
        pltpu.SemaphoreType.DMA,
    ],
    compiler_params=_sc_params,
)
def _sc_pass2(hA, offA, pB, hB, offB, pA,
              mA, dA, SA, mB, dB, SB,
              buf, offs_v, p_v, S_v, d_v, m_v, ebuf, mstage, dstage,
              offs_s, sem):
  wid = lax.axis_index("s") * NC + lax.axis_index("c")
  iota = _lane_iota()

  for h, off_hbm, p_hbm, m_out, d_out, S_out in (
      (hA, offA, pB, mA, dA, SA),
      (hB, offB, pA, mB, dB, SB)):
    _stage_offsets(off_hbm, offs_v, offs_s)
    pltpu.sync_copy(p_hbm, p_v)
    z = jnp.zeros((L,), _f32)

    def zbody(s, _):
      for c in range(CPR):
        S_v[s, pl.ds(c * L, L)] = z
      d_v[pl.ds(s * L, L)] = z
      m_v[pl.ds(s * L, L)] = jnp.full((L,), NEG, _f32)
      return 0

    lax.fori_loop(0, SEG, zbody, 0)
    base0 = wid * RPW
    pltpu.make_async_copy(h.at[pl.ds(base0, CHUNK)], buf.at[0], sem).start()

    def chunk_body(k, _):
      par = lax.bitwise_and(k, 1)
      base = base0 + k * CHUNK
      pltpu.make_async_copy(h.at[pl.ds(0, CHUNK)], buf.at[0], sem).wait()

      @pl.when(k + 1 < NCHUNK)
      def _():
        pltpu.make_async_copy(
            h.at[pl.ds(base + CHUNK, CHUNK)],
            buf.at[lax.bitwise_and(k + 1, 1)], sem).start()

      def seg_body(s, _):
        lo = jnp.clip(offs_s[s] - base, 0, CHUNK)
        hi = jnp.clip(offs_s[s + 1] - base, 0, CHUNK)

        @pl.when(hi > lo)
        def _():
          prow = [p_v[s, pl.ds(c * L, L)] for c in range(CPR)]

          def dot_row(r, cmax):
            hrow = [buf[par, r, pl.ds(c * L, L)] for c in range(CPR)]
            t = [hrow[c] * prow[c] for c in range(CPR)]
            while len(t) > 1:
              t = [t[i] + t[i + 1] for i in range(0, len(t) - 1, 2)] + (
                  [t[-1]] if len(t) & 1 else [])
            e = _allsum(t[0])
            ebuf[pl.ds(r * L, L)] = e
            return jnp.maximum(cmax, e)

          def dot_pair(j, cmax):
            r0 = lo + j * 2
            return dot_row(r0 + 1, dot_row(r0, cmax))

          n2 = lax.bitwise_and(hi - lo, -2)
          cmax = lax.fori_loop(0, lax.shift_right_logical(n2, 1), dot_pair,
                               jnp.full((L,), NEG, _f32))
          cmax = lax.fori_loop(lo + n2, hi, dot_row, cmax)

          m_old = m_v[pl.ds(s * L, L)]
          m_new = jnp.maximum(m_old, cmax)
          corr = jnp.exp(m_old - m_new)
          m_v[pl.ds(s * L, L)] = m_new
          for c in range(CPR):
            S_v[s, pl.ds(c * L, L)] *= corr

          def wsum_grp(g, carry):
            dsum = carry[0]
            S = list(carry[1:])
            rb = lo + g * L
            e16 = plsc.load_gather(ebuf, [(rb + iota) * L])
            e16 = jnp.where(rb + iota < hi, e16, NEG)
            p16 = jnp.exp(e16 - m_new)
            dsum = dsum + p16
            for r in range(L):
              prb = _shuffle(p16, jnp.full((L,), r, _i32))
              hr = jnp.minimum(rb + r, CHUNK - 1)
              for c in range(CPR):
                S[c] = S[c] + buf[par, hr, pl.ds(c * L, L)] * prb
            return (dsum,) + tuple(S)

          ng = lax.shift_right_logical(hi - lo + (L - 1), 4)
          init = (jnp.zeros((L,), _f32),) + tuple(
              S_v[s, pl.ds(c * L, L)] for c in range(CPR))
          res = lax.fori_loop(0, ng, wsum_grp, init)
          d_v[pl.ds(s * L, L)] = (
              d_v[pl.ds(s * L, L)] * corr + _allsum(res[0]))
          for c in range(CPR):
            S_v[s, pl.ds(c * L, L)] = res[1 + c]

        return 0

      lax.fori_loop(0, SEG, seg_body, 0)
      return 0

    lax.fori_loop(0, NCHUNK, chunk_body, 0)

    mstage[pl.ds(0, L)] = plsc.load_gather(m_v, [iota * (L + 1)])
    dstage[pl.ds(0, L)] = plsc.load_gather(d_v, [iota * (L + 1)])
    pltpu.sync_copy(mstage, m_out.at[wid])
    pltpu.sync_copy(dstage, d_out.at[wid])
    pltpu.sync_copy(S_v, S_out.at[wid])


def _tc_merge_body(mA, dA, SA, mB, dB, SB, outA, outB):
  for m_r, d_r, S_r, out in ((mA, dA, SA, outA), (mB, dB, SB, outB)):
    m = m_r[...]
    d = d_r[...]
    S = S_r[...]
    M = jnp.max(m, axis=0, keepdims=True)
    scale = jnp.exp(m - M)
    den = jnp.sum(d * scale, axis=0)[:, None]
    num = jnp.sum(S * scale[:, :, None], axis=0)
    out[...] = jnp.where(den > 0.0, num / den, 0.0)


def _tc_merge(mA, dA, SA, mB, dB, SB):
  return pl.pallas_call(
      _tc_merge_body,
      out_shape=[jax.ShapeDtypeStruct((SEG, D), _f32),
                 jax.ShapeDtypeStruct((SEG, D), _f32)],
  )(mA, dA, SA, mB, dB, SB)


def kernel(h_mol_A, herb_batch_A, h_mol_B, herb_batch_B, W_attn):
  qs = jnp.arange(SEG + 1, dtype=_i32)
  offA = jnp.searchsorted(herb_batch_A, qs).astype(_i32)
  offB = jnp.searchsorted(herb_batch_B, qs).astype(_i32)
  cntA = (offA[1:] - offA[:-1]).astype(_f32)[:, None]
  cntB = (offB[1:] - offB[:-1]).astype(_f32)[:, None]
  offA32 = jnp.concatenate([offA, jnp.full((32 - SEG - 1,), N, _i32)])
  offB32 = jnp.concatenate([offB, jnp.full((32 - SEG - 1,), N, _i32)])

  psA, psB = _sc_pass1(h_mol_A, offA32, h_mol_B, offB32)
  projA, projB = _tc_proj(psA, psB, cntA, cntB, W_attn)
  mA, dA, SA, mB, dB, SB = _sc_pass2(
      h_mol_A, offA32, projB, h_mol_B, offB32, projA)
  outA, outB = _tc_merge(mA, dA, SA, mB, dB, SB)
  return (outA, outB)


# speedup vs baseline: 1.1967x; 1.1967x over previous
import functools

import jax
import jax.numpy as jnp
from jax import lax
from jax.experimental import pallas as pl
from jax.experimental.pallas import tpu as pltpu
from jax.experimental.pallas import tpu_sc as plsc

D = 256
SEG = 16
N = 32768
NC = 2
NS = 16
NW = NC * NS
RPW = N // NW
CHUNK = 128
NCHUNK = RPW // CHUNK
L = 16
CPR = D // L
NEG = -1e30

_mesh = plsc.VectorSubcoreMesh(
    core_axis_name="c", subcore_axis_name="s", num_cores=NC, num_subcores=NS)

_f32 = jnp.float32
_i32 = jnp.int32

_sc_params = pltpu.CompilerParams(needs_layout_passes=False)


def _lane_iota():
  return lax.iota(_i32, L)


_GATHER_DNUMS = lax.GatherDimensionNumbers(
    offset_dims=(), collapsed_slice_dims=(0,), start_index_map=(0,))


def _shuffle(vec, idx):
  return lax.gather(vec, idx[:, None], _GATHER_DNUMS, (1,),
                    mode=lax.GatherScatterMode.PROMISE_IN_BOUNDS)


def _allsum(vec):
  iota = _lane_iota()
  for k in (1, 2, 4, 8):
    vec = vec + _shuffle(vec, jnp.bitwise_xor(iota, k))
  return vec


def _stage_offsets(off_hbm, offs_v, offs_s):
  pltpu.sync_copy(off_hbm, offs_v)
  v0 = offs_v[pl.ds(0, L)]
  v1 = offs_v[pl.ds(L, L)]
  for s in range(L):
    offs_s[s] = v0[s]
  offs_s[L] = v1[0]


@functools.partial(
    pl.kernel,
    out_type=[jax.ShapeDtypeStruct((NW, SEG, D), _f32),
              jax.ShapeDtypeStruct((NW, SEG, D), _f32)],
    mesh=_mesh,
    scratch_types=[
        pltpu.VMEM((2, CHUNK, D), _f32),
        pltpu.VMEM((32,), _i32),
        pltpu.VMEM((SEG, D), _f32),
        pltpu.SMEM((SEG + 1,), _i32),
        pltpu.SemaphoreType.DMA,
    ],
    compiler_params=_sc_params,
)
def _sc_pass1(hA, offA, hB, offB, outA, outB, buf, offs_v, acc, offs_s, sem):
  wid = lax.axis_index("s") * NC + lax.axis_index("c")

  for h, off_hbm, out in ((hA, offA, outA), (hB, offB, outB)):
    _stage_offsets(off_hbm, offs_v, offs_s)
    z = jnp.zeros((L,), _f32)

    def zbody(s, _):
      for c in range(CPR):
        acc[s, pl.ds(c * L, L)] = z
      return 0

    lax.fori_loop(0, SEG, zbody, 0)
    base0 = wid * RPW
    pltpu.make_async_copy(h.at[pl.ds(base0, CHUNK)], buf.at[0], sem).start()

    def chunk_body(k, _):
      par = lax.bitwise_and(k, 1)
      base = base0 + k * CHUNK
      pltpu.make_async_copy(h.at[pl.ds(0, CHUNK)], buf.at[0], sem).wait()

      @pl.when(k + 1 < NCHUNK)
      def _():
        pltpu.make_async_copy(
            h.at[pl.ds(base + CHUNK, CHUNK)],
            buf.at[lax.bitwise_and(k + 1, 1)], sem).start()

      def seg_body(s, _):
        lo = jnp.clip(offs_s[s] - base, 0, CHUNK)
        hi = jnp.clip(offs_s[s + 1] - base, 0, CHUNK)

        @pl.when(hi > lo)
        def _():
          def row_body(r, carry):
            return tuple(
                carry[c] + buf[par, r, pl.ds(c * L, L)] for c in range(CPR))

          def quad_body(j, carry):
            r0 = lo + j * 4
            for u in range(4):
              carry = row_body(r0 + u, carry)
            return carry

          n4 = lax.bitwise_and(hi - lo, -4)
          zeros = tuple(jnp.zeros((L,), _f32) for _ in range(CPR))
          tot = lax.fori_loop(0, lax.shift_right_logical(n4, 2),
                              quad_body, zeros)
          tot = lax.fori_loop(lo + n4, hi, row_body, tot)
          for c in range(CPR):
            acc[s, pl.ds(c * L, L)] += tot[c]

        return 0

      lax.fori_loop(0, SEG, seg_body, 0)
      return 0

    lax.fori_loop(0, NCHUNK, chunk_body, 0)
    pltpu.sync_copy(acc, out.at[wid])


def _tc_proj_body(psA, psB, cA, cB, w, projA, projB):
  wmat = w[...]
  for ps, cnt, out in ((psA, cA, projA), (psB, cB, projB)):
    sums = jnp.sum(ps[...], axis=0)
    mean = sums / jnp.maximum(cnt[...], 1.0)
    out[...] = jnp.dot(mean, wmat, preferred_element_type=_f32)


def _tc_proj(psA, psB, cA, cB, w):
  return pl.pallas_call(
      _tc_proj_body,
      out_shape=[jax.ShapeDtypeStruct((SEG, D), _f32),
                 jax.ShapeDtypeStruct((SEG, D), _f32)],
  )(psA, psB, cA, cB, w)


@functools.partial(
    pl.kernel,
    out_type=[jax.ShapeDtypeStruct((NW, SEG), _f32),
              jax.ShapeDtypeStruct((NW, SEG), _f32),
              jax.ShapeDtypeStruct((NW, SEG, D), _f32),
              jax.ShapeDtypeStruct((NW, SEG), _f32),
              jax.ShapeDtypeStruct((NW, SEG), _f32),
              jax.ShapeDtypeStruct((NW, SEG, D), _f32)],
    mesh=_mesh,
    scratch_types=[
        pltpu.VMEM((2, CHUNK, D), _f32),
        pltpu.VMEM((32,), _i32),
        pltpu.VMEM((SEG, D), _f32),
        pltpu.VMEM((SEG, D), _f32),
        pltpu.VMEM((SEG * L,), _f32),
        pltpu.VMEM((SEG * L,), _f32),
        pltpu.VMEM((L,), _f32),
        pltpu.VMEM((L,), _f32),
        pltpu.SMEM((SEG + 1,), _i32),
        pltpu.SemaphoreType.DMA,
    ],
    compiler_params=_sc_params,
)
def _sc_pass2(hA, offA, pB, hB, offB, pA,
              mA, dA, SA, mB, dB, SB,
              buf, offs_v, p_v, S_v, d_v, m_v, mstage, dstage,
              offs_s, sem):
  wid = lax.axis_index("s") * NC + lax.axis_index("c")
  iota = _lane_iota()

  for h, off_hbm, p_hbm, m_out, d_out, S_out in (
      (hA, offA, pB, mA, dA, SA),
      (hB, offB, pA, mB, dB, SB)):
    _stage_offsets(off_hbm, offs_v, offs_s)
    pltpu.sync_copy(p_hbm, p_v)
    z = jnp.zeros((L,), _f32)

    def zbody(s, _):
      for c in range(CPR):
        S_v[s, pl.ds(c * L, L)] = z
      d_v[pl.ds(s * L, L)] = z
      m_v[pl.ds(s * L, L)] = jnp.full((L,), NEG, _f32)
      return 0

    lax.fori_loop(0, SEG, zbody, 0)
    base0 = wid * RPW
    pltpu.make_async_copy(h.at[pl.ds(base0, CHUNK)], buf.at[0], sem).start()

    def chunk_body(k, _):
      par = lax.bitwise_and(k, 1)
      base = base0 + k * CHUNK
      pltpu.make_async_copy(h.at[pl.ds(0, CHUNK)], buf.at[0], sem).wait()

      @pl.when(k + 1 < NCHUNK)
      def _():
        pltpu.make_async_copy(
            h.at[pl.ds(base + CHUNK, CHUNK)],
            buf.at[lax.bitwise_and(k + 1, 1)], sem).start()

      def seg_body(s, _):
        lo = jnp.clip(offs_s[s] - base, 0, CHUNK)
        hi = jnp.clip(offs_s[s + 1] - base, 0, CHUNK)

        @pl.when(hi > lo)
        def _():
          prow = [p_v[s, pl.ds(c * L, L)] for c in range(CPR)]

          def row_body(r, carry):
            m = carry[0]
            d = carry[1]
            S = list(carry[2:])
            hrow = [buf[par, r, pl.ds(c * L, L)] for c in range(CPR)]
            t = [hrow[c] * prow[c] for c in range(CPR)]
            while len(t) > 1:
              t = [t[i] + t[i + 1] for i in range(0, len(t) - 1, 2)] + (
                  [t[-1]] if len(t) & 1 else [])
            e = _allsum(t[0])
            m_new = jnp.maximum(m, e)
            corr = jnp.exp(m - m_new)
            p = jnp.exp(e - m_new)
            d = d * corr + p
            for c in range(CPR):
              S[c] = S[c] * corr + hrow[c] * p
            return (m_new, d) + tuple(S)

          def pair_body(j, carry):
            r0 = lo + j * 2
            carry = row_body(r0, carry)
            return row_body(r0 + 1, carry)

          init = (m_v[pl.ds(s * L, L)], d_v[pl.ds(s * L, L)]) + tuple(
              S_v[s, pl.ds(c * L, L)] for c in range(CPR))
          n2 = lax.bitwise_and(hi - lo, -2)
          res = lax.fori_loop(0, lax.shift_right_logical(n2, 1),
                              pair_body, init)
          res = lax.fori_loop(lo + n2, hi, row_body, res)
          m_v[pl.ds(s * L, L)] = res[0]
          d_v[pl.ds(s * L, L)] = res[1]
          for c in range(CPR):
            S_v[s, pl.ds(c * L, L)] = res[2 + c]

        return 0

      lax.fori_loop(0, SEG, seg_body, 0)
      return 0

    lax.fori_loop(0, NCHUNK, chunk_body, 0)

    mstage[pl.ds(0, L)] = plsc.load_gather(m_v, [iota * (L + 1)])
    dstage[pl.ds(0, L)] = plsc.load_gather(d_v, [iota * (L + 1)])
    pltpu.sync_copy(mstage, m_out.at[wid])
    pltpu.sync_copy(dstage, d_out.at[wid])
    pltpu.sync_copy(S_v, S_out.at[wid])


def _tc_merge_body(mA, dA, SA, mB, dB, SB, outA, outB):
  for m_r, d_r, S_r, out in ((mA, dA, SA, outA), (mB, dB, SB, outB)):
    m = m_r[...]
    d = d_r[...]
    S = S_r[...]
    M = jnp.max(m, axis=0, keepdims=True)
    scale = jnp.exp(m - M)
    den = jnp.sum(d * scale, axis=0)[:, None]
    num = jnp.sum(S * scale[:, :, None], axis=0)
    out[...] = jnp.where(den > 0.0, num / den, 0.0)


def _tc_merge(mA, dA, SA, mB, dB, SB):
  return pl.pallas_call(
      _tc_merge_body,
      out_shape=[jax.ShapeDtypeStruct((SEG, D), _f32),
                 jax.ShapeDtypeStruct((SEG, D), _f32)],
  )(mA, dA, SA, mB, dB, SB)


def kernel(h_mol_A, herb_batch_A, h_mol_B, herb_batch_B, W_attn):
  qs = jnp.arange(SEG + 1, dtype=_i32)
  offA = jnp.searchsorted(herb_batch_A, qs).astype(_i32)
  offB = jnp.searchsorted(herb_batch_B, qs).astype(_i32)
  cntA = (offA[1:] - offA[:-1]).astype(_f32)[:, None]
  cntB = (offB[1:] - offB[:-1]).astype(_f32)[:, None]
  offA32 = jnp.concatenate([offA, jnp.full((32 - SEG - 1,), N, _i32)])
  offB32 = jnp.concatenate([offB, jnp.full((32 - SEG - 1,), N, _i32)])

  psA, psB = _sc_pass1(h_mol_A, offA32, h_mol_B, offB32)
  projA, projB = _tc_proj(psA, psB, cntA, cntB, W_attn)
  mA, dA, SA, mB, dB, SB = _sc_pass2(
      h_mol_A, offA32, projB, h_mol_B, offB32, projA)
  outA, outB = _tc_merge(mA, dA, SA, mB, dB, SB)
  return (outA, outB)
